# baseline (speedup 1.0000x reference)
import jax
import jax.numpy as jnp
from jax import lax
from jax.experimental import pallas as pl
from jax.experimental.pallas import tpu as pltpu
from jax.experimental.pallas import tpu_sc as plsc

VOCAB = 49408
MAX_POS = 77
EMBED_DIM = 768
BATCH = 4096
SEQ = 77
SEQ_PAD = 80
HALF = SEQ_PAD // 2
MAIN_B = 32
TAIL = 8

NUM_CORES = 2
NUM_SUBCORES = 16
NUM_WORKERS = NUM_CORES * NUM_SUBCORES
SEQS_PER_WORKER = BATCH // NUM_WORKERS
IDX_PER_WORKER = SEQS_PER_WORKER * SEQ_PAD
NUM_CHUNKS = 2 * SEQS_PER_WORKER
GROUP = 16
LANES = 16
D_VECS = EMBED_DIM // LANES


def _body(ids_hbm, tok_hbm, pos_hbm, out_hbm, tail_hbm,
          idx_g, rows0, rows1, pos_v, sg0, sg1, ss0, ss1):
    wid = lax.axis_index("s") * NUM_CORES + lax.axis_index("c")
    q_base = wid * SEQS_PER_WORKER

    rows = (rows0, rows1)
    sg = (sg0, sg1)
    ss = (ss0, ss1)

    def stage_group(tq):
        pltpu.sync_copy(
            ids_hbm.at[pl.ds(wid * IDX_PER_WORKER + tq * SEQ_PAD,
                             GROUP * SEQ_PAD)], idx_g)

    pltpu.sync_copy(pos_hbm, pos_v)
    stage_group(0)

    def gather_start(tmod, h, b):
        idx = idx_g.at[pl.ds(tmod * SEQ_PAD + h * HALF, HALF)]
        pltpu.async_copy(tok_hbm.at[idx], rows[b], sg[b])

    gather_start(0, 0, 0)

    def seq_body(t, _):
        tmod = lax.rem(t, GROUP)
        for k in range(2):
            c = 2 * t + k
            b = k
            q = q_base + t

            pltpu.make_async_copy(
                tok_hbm.at[idx_g.at[pl.ds(0, HALF)]], rows[b], sg[b]).wait()

            @pl.when(c > 0)
            def _():
                if k == 0:
                    pltpu.make_async_copy(
                        rows[1 - b].at[pl.ds(0, MAIN_B)],
                        out_hbm.at[0, pl.ds(0, MAIN_B), :], ss[1 - b]).wait()
                    pltpu.make_async_copy(
                        rows[1 - b].at[pl.ds(MAIN_B, TAIL)],
                        tail_hbm.at[0], ss[1 - b]).wait()
                else:
                    pltpu.make_async_copy(
                        rows[1 - b], out_hbm.at[0, pl.ds(0, HALF), :],
                        ss[1 - b]).wait()

            @pl.when(c + 1 < NUM_CHUNKS)
            def _():
                if k == 0:
                    gather_start(tmod, 1, 1)
                else:
                    @pl.when(tmod == GROUP - 1)
                    def _():
                        stage_group(t + 1)
                    gather_start(lax.rem(t + 1, GROUP), 0, 0)

            def row_body(i, _):
                vals = [pos_v[k * HALF + i, pl.ds(j * LANES, LANES)]
                        for j in range(D_VECS)]
                for j in range(D_VECS):
                    plsc.addupdate(rows[b].at[i, pl.ds(j * LANES, LANES)],
                                   vals[j])
                return 0

            lax.fori_loop(0, HALF, row_body, 0)

            if k == 0:
                pltpu.async_copy(rows[b], out_hbm.at[q, pl.ds(0, HALF), :],
                                 ss[b])
            else:
                pltpu.async_copy(rows[b].at[pl.ds(0, MAIN_B)],
                                 out_hbm.at[q, pl.ds(HALF, MAIN_B), :], ss[b])
                pltpu.async_copy(rows[b].at[pl.ds(MAIN_B, TAIL)],
                                 tail_hbm.at[q], ss[b])
        return 0

    lax.fori_loop(0, SEQS_PER_WORKER, seq_body, 0)

    pltpu.make_async_copy(rows[1].at[pl.ds(0, MAIN_B)],
                          out_hbm.at[0, pl.ds(0, MAIN_B), :], ss[1]).wait()
    pltpu.make_async_copy(rows[1].at[pl.ds(MAIN_B, TAIL)],
                          tail_hbm.at[0], ss[1]).wait()


@jax.jit
def kernel(input_ids, token_table, position_table):
    ids_pad = jnp.pad(input_ids.astype(jnp.int32),
                      ((0, 0), (0, SEQ_PAD - SEQ))).reshape(BATCH * SEQ_PAD)
    pos_pad = jnp.pad(position_table, ((0, SEQ_PAD - SEQ), (0, 0)))
    mesh = plsc.VectorSubcoreMesh(core_axis_name="c", subcore_axis_name="s")
    out, tail = pl.kernel(
        _body,
        out_type=(
            jax.ShapeDtypeStruct((BATCH, SEQ, EMBED_DIM), jnp.float32),
            jax.ShapeDtypeStruct((BATCH, TAIL, EMBED_DIM), jnp.float32),
        ),
        mesh=mesh,
        scratch_types=[
            pltpu.VMEM((GROUP * SEQ_PAD,), jnp.int32),
            pltpu.VMEM((HALF, EMBED_DIM), jnp.float32),
            pltpu.VMEM((HALF, EMBED_DIM), jnp.float32),
            pltpu.VMEM((SEQ_PAD, EMBED_DIM), jnp.float32),
            pltpu.SemaphoreType.DMA,
            pltpu.SemaphoreType.DMA,
            pltpu.SemaphoreType.DMA,
            pltpu.SemaphoreType.DMA,
        ],
    )(ids_pad, token_table, pos_pad)
    return lax.dynamic_update_slice(out, tail[:, :SEQ - 72, :], (0, 72, 0))


# speedup vs baseline: 3.4216x; 3.4216x over previous
import jax
import jax.numpy as jnp
from jax import lax
from jax.experimental import pallas as pl
from jax.experimental.pallas import tpu as pltpu
from jax.experimental.pallas import tpu_sc as plsc

VOCAB = 49408
MAX_POS = 77
EMBED_DIM = 768
BATCH = 4096
SEQ = 77

NUM_CORES = 2
NUM_SUBCORES = 16
NUM_WORKERS = NUM_CORES * NUM_SUBCORES
B_PER_WORKER = BATCH // NUM_WORKERS
NB = 64
NUM_CHUNKS = 2 * SEQ
LANES = 16
D_VECS = EMBED_DIM // LANES


def _body(idsT_hbm, tok_hbm, pos_hbm, out_hbm,
          idx0, idx1, posb0, posb1, rows0, rows1,
          sg0, sg1, ss0, ss1, st):
    wid = lax.axis_index("s") * NUM_CORES + lax.axis_index("c")
    b_base = wid * B_PER_WORKER

    idx = (idx0, idx1)
    posb = (posb0, posb1)
    rows = (rows0, rows1)
    sg = (sg0, sg1)
    ss = (ss0, ss1)

    def stage_async(s, p):
        pltpu.async_copy(
            idsT_hbm.at[pl.ds(s * BATCH + b_base, B_PER_WORKER)], idx[p], st)
        pltpu.async_copy(
            pos_hbm.at[pl.ds(s * EMBED_DIM, EMBED_DIM)], posb[p], st)

    def stage_wait(p):
        pltpu.make_async_copy(
            idsT_hbm.at[pl.ds(0, B_PER_WORKER)], idx[p], st).wait()
        pltpu.make_async_copy(
            pos_hbm.at[pl.ds(0, EMBED_DIM)], posb[p], st).wait()

    def gather_start(p, h, b):
        pltpu.async_copy(tok_hbm.at[idx[p].at[pl.ds(h * NB, NB)]],
                         rows[b], sg[b])

    def add_and_scatter(s, b, vals):
        def row_body(i, _):
            for j in range(D_VECS):
                plsc.addupdate(rows[b].at[i, pl.ds(j * LANES, LANES)],
                               vals[j])
            return 0

        lax.fori_loop(0, NB, row_body, 0)
        pltpu.async_copy(rows[b],
                         out_hbm.at[s, pl.ds(b_base + b * NB, NB), :], ss[b])

    def wait_gather(b):
        pltpu.make_async_copy(
            tok_hbm.at[idx0.at[pl.ds(0, NB)]], rows[b], sg[b]).wait()

    def wait_scatter(b):
        pltpu.make_async_copy(
            rows[b], out_hbm.at[0, pl.ds(0, NB), :], ss[b]).wait()

    stage_async(0, 0)
    stage_wait(0)
    gather_start(0, 0, 0)

    def pair_body(u, _):
        for su in range(2):
            s = 2 * u + su
            p = su
            vals = [posb[p][pl.ds(j * LANES, LANES)] for j in range(D_VECS)]

            wait_gather(0)

            @pl.when(s > 0)
            def _():
                wait_scatter(1)

            stage_async(s + 1, 1 - p)
            gather_start(p, 1, 1)
            add_and_scatter(s, 0, vals)

            wait_gather(1)
            wait_scatter(0)
            stage_wait(1 - p)
            gather_start(1 - p, 0, 0)
            add_and_scatter(s, 1, vals)
        return 0

    lax.fori_loop(0, (SEQ - 1) // 2, pair_body, 0)

    vals = [posb0[pl.ds(j * LANES, LANES)] for j in range(D_VECS)]
    wait_gather(0)
    wait_scatter(1)
    gather_start(0, 1, 1)
    add_and_scatter(SEQ - 1, 0, vals)
    wait_gather(1)
    wait_scatter(0)
    add_and_scatter(SEQ - 1, 1, vals)
    wait_scatter(1)


@jax.jit
def kernel(input_ids, token_table, position_table):
    idsT = input_ids.astype(jnp.int32).T.reshape(SEQ * BATCH)
    pos_flat = position_table.reshape(MAX_POS * EMBED_DIM)
    mesh = plsc.VectorSubcoreMesh(core_axis_name="c", subcore_axis_name="s")
    outT = pl.kernel(
        _body,
        out_type=jax.ShapeDtypeStruct((SEQ, BATCH, EMBED_DIM), jnp.float32),
        mesh=mesh,
        scratch_types=[
            pltpu.VMEM((B_PER_WORKER,), jnp.int32),
            pltpu.VMEM((B_PER_WORKER,), jnp.int32),
            pltpu.VMEM((EMBED_DIM,), jnp.float32),
            pltpu.VMEM((EMBED_DIM,), jnp.float32),
            pltpu.VMEM((NB, EMBED_DIM), jnp.float32),
            pltpu.VMEM((NB, EMBED_DIM), jnp.float32),
            pltpu.SemaphoreType.DMA,
            pltpu.SemaphoreType.DMA,
            pltpu.SemaphoreType.DMA,
            pltpu.SemaphoreType.DMA,
            pltpu.SemaphoreType.DMA,
        ],
    )(idsT, token_table, pos_flat)
    return jnp.transpose(outT, (1, 0, 2))


# speedup vs baseline: 3.4321x; 1.0031x over previous
import jax
import jax.numpy as jnp
from jax import lax
from jax.experimental import pallas as pl
from jax.experimental.pallas import tpu as pltpu
from jax.experimental.pallas import tpu_sc as plsc

VOCAB = 49408
MAX_POS = 77
EMBED_DIM = 768
BATCH = 4096
SEQ = 77

NUM_CORES = 2
NUM_SUBCORES = 16
NUM_WORKERS = NUM_CORES * NUM_SUBCORES
B_PER_WORKER = BATCH // NUM_WORKERS
NB = 64
NUM_CHUNKS = 2 * SEQ
LANES = 16
D_VECS = EMBED_DIM // LANES


def _body(idsT_hbm, tok_hbm, pos_hbm, out_hbm,
          idx0, idx1, posb0, posb1, rows0, rows1,
          sg0, sg1, ss0, ss1, st):
    wid = lax.axis_index("s") * NUM_CORES + lax.axis_index("c")
    b_base = wid * B_PER_WORKER

    idx = (idx0, idx1)
    posb = (posb0, posb1)
    rows = (rows0, rows1)
    sg = (sg0, sg1)
    ss = (ss0, ss1)

    def stage_async(s, p):
        pltpu.async_copy(
            idsT_hbm.at[pl.ds(s * BATCH + b_base, B_PER_WORKER)], idx[p], st)
        pltpu.async_copy(
            pos_hbm.at[pl.ds(s * EMBED_DIM, EMBED_DIM)], posb[p], st)

    def stage_wait(p):
        pltpu.make_async_copy(
            idsT_hbm.at[pl.ds(0, B_PER_WORKER)], idx[p], st).wait()
        pltpu.make_async_copy(
            pos_hbm.at[pl.ds(0, EMBED_DIM)], posb[p], st).wait()

    def gather_start(p, h, b):
        pltpu.async_copy(tok_hbm.at[idx[p].at[pl.ds(h * NB, NB)]],
                         rows[b], sg[b])

    HB = NB // 2

    def add_and_scatter(s, b, vals):
        def row_body(i, _):
            for j in range(D_VECS):
                plsc.addupdate(rows[b].at[i, pl.ds(j * LANES, LANES)],
                               vals[j])
            return 0

        for h in range(2):
            lax.fori_loop(h * HB, (h + 1) * HB, row_body, 0)
            pltpu.async_copy(
                rows[b].at[pl.ds(h * HB, HB)],
                out_hbm.at[s, pl.ds(b_base + b * NB + h * HB, HB), :], ss[b])

    def wait_gather(b):
        pltpu.make_async_copy(
            tok_hbm.at[idx0.at[pl.ds(0, NB)]], rows[b], sg[b]).wait()

    def wait_scatter(b):
        for _ in range(2):
            pltpu.make_async_copy(
                rows[b].at[pl.ds(0, HB)],
                out_hbm.at[0, pl.ds(0, HB), :], ss[b]).wait()

    stage_async(0, 0)
    stage_wait(0)
    gather_start(0, 0, 0)

    def pair_body(u, _):
        for su in range(2):
            s = 2 * u + su
            p = su
            vals = [posb[p][pl.ds(j * LANES, LANES)] for j in range(D_VECS)]

            wait_gather(0)

            @pl.when(s > 0)
            def _():
                wait_scatter(1)

            stage_async(s + 1, 1 - p)
            gather_start(p, 1, 1)
            add_and_scatter(s, 0, vals)

            wait_gather(1)
            wait_scatter(0)
            stage_wait(1 - p)
            gather_start(1 - p, 0, 0)
            add_and_scatter(s, 1, vals)
        return 0

    lax.fori_loop(0, (SEQ - 1) // 2, pair_body, 0)

    vals = [posb0[pl.ds(j * LANES, LANES)] for j in range(D_VECS)]
    wait_gather(0)
    wait_scatter(1)
    gather_start(0, 1, 1)
    add_and_scatter(SEQ - 1, 0, vals)
    wait_gather(1)
    wait_scatter(0)
    add_and_scatter(SEQ - 1, 1, vals)
    wait_scatter(1)


@jax.jit
def kernel(input_ids, token_table, position_table):
    idsT = input_ids.astype(jnp.int32).T.reshape(SEQ * BATCH)
    pos_flat = position_table.reshape(MAX_POS * EMBED_DIM)
    mesh = plsc.VectorSubcoreMesh(core_axis_name="c", subcore_axis_name="s")
    outT = pl.kernel(
        _body,
        out_type=jax.ShapeDtypeStruct((SEQ, BATCH, EMBED_DIM), jnp.float32),
        mesh=mesh,
        scratch_types=[
            pltpu.VMEM((B_PER_WORKER,), jnp.int32),
            pltpu.VMEM((B_PER_WORKER,), jnp.int32),
            pltpu.VMEM((EMBED_DIM,), jnp.float32),
            pltpu.VMEM((EMBED_DIM,), jnp.float32),
            pltpu.VMEM((NB, EMBED_DIM), jnp.float32),
            pltpu.VMEM((NB, EMBED_DIM), jnp.float32),
            pltpu.SemaphoreType.DMA,
            pltpu.SemaphoreType.DMA,
            pltpu.SemaphoreType.DMA,
            pltpu.SemaphoreType.DMA,
            pltpu.SemaphoreType.DMA,
        ],
    )(idsT, token_table, pos_flat)
    return jnp.transpose(outT, (1, 0, 2))
